# Initial kernel scaffold; baseline (speedup 1.0000x reference)
import jax
import jax.numpy as jnp
from jax.experimental import pallas as pl


def kernel(user_id, gender, age, occup, zipc, item_id, year, user_table, gender_table, age_table, occup_table, zip_table, item_table, year_table, W1, b1, W2, b2, W3, b3):
    raise NotImplementedError("write your pallas kernel here")


# speedup vs baseline: 1.0168x; 1.0168x over previous
import functools

import jax
import jax.numpy as jnp
from jax import lax
from jax.experimental import pallas as pl
from jax.experimental.pallas import tpu as pltpu
from jax.experimental.pallas import tpu_sc as plsc

B = 16384
EMB = 128
H1P = 128
H2P = 128

SIZES = (6041, 2, 7, 21, 3439, 3884, 81)
PADS = (6048, 8, 8, 24, 3440, 3888, 88)
OFFS = (0, 6048, 6056, 6064, 6088, 9528, 13416)
RTOT = 13504

NC = 2
NS = 16
NW = NC * NS
BPW = B // NW
CH = 128
NCH = BPW // CH


def _proj_body(t_ref, w_ref, p_ref):
    for f in range(7):
        o, n = OFFS[f], PADS[f]
        p_ref[o:o + n, :] = jnp.dot(
            t_ref[o:o + n, :], w_ref[f * EMB:(f + 1) * EMB, :],
            preferred_element_type=jnp.float32)


def _tail_body(h_ref, b1_ref, w2_ref, b2_ref, w3_ref, o_ref):
    x = jnp.maximum(h_ref[...] + b1_ref[...], 0.0)
    h2 = jnp.maximum(
        jnp.dot(x, w2_ref[...], preferred_element_type=jnp.float32)
        + b2_ref[...], 0.0)
    o_ref[...] = jnp.sum(h2 * w3_ref[...], axis=1)


@functools.cache
def _make_gather_sum():
    mesh = plsc.VectorSubcoreMesh(core_axis_name="c", subcore_axis_name="s",
                                  num_cores=NC, num_subcores=NS)
    return functools.partial(
        pl.kernel,
        out_type=jax.ShapeDtypeStruct((B, H1P), jnp.float32),
        mesh=mesh,
        scratch_types=[
            pltpu.VMEM((CH,), jnp.int32),
            pltpu.VMEM((CH, H1P), jnp.float32),
            pltpu.VMEM((CH, H1P), jnp.float32),
            pltpu.SemaphoreType.DMA,
        ],
    )(_gather_sum_body)


def _gather_sum_body(p_hbm, idx_hbm, out_hbm, idx_v, acc_v, tmp_v, sem):
    wid = lax.axis_index("s") * NC + lax.axis_index("c")
    base = wid * BPW
    for blk in range(NCH):
        rb = base + blk * CH
        pltpu.sync_copy(idx_hbm.at[pl.ds(rb, CH)], idx_v)
        pltpu.async_copy(p_hbm.at[idx_v], acc_v, sem).wait()
        for f in range(1, 7):
            pltpu.sync_copy(idx_hbm.at[pl.ds(f * B + rb, CH)], idx_v)
            pltpu.async_copy(p_hbm.at[idx_v], tmp_v, sem).wait()

            def _add_rows(r, _):
                for rr in range(4):
                    for c in range(H1P // 16):
                        sl = pl.ds(c * 16, 16)
                        plsc.addupdate(acc_v.at[r * 4 + rr, sl],
                                       tmp_v[r * 4 + rr, sl])
                return 0

            lax.fori_loop(0, CH // 4, _add_rows, 0)
        pltpu.sync_copy(acc_v, out_hbm.at[pl.ds(rb, CH)])


def kernel(user_id, gender, age, occup, zipc, item_id, year,
           user_table, gender_table, age_table, occup_table, zip_table,
           item_table, year_table, W1, b1, W2, b2, W3, b3):
    f32 = jnp.float32
    i32 = jnp.int32

    tables = (user_table, gender_table, age_table, occup_table, zip_table,
              item_table, year_table)
    stacked = jnp.concatenate(
        [jnp.pad(t, ((0, p - s), (0, 0)))
         for t, s, p in zip(tables, SIZES, PADS)], axis=0)
    ids = (user_id.astype(i32) - 1, gender.astype(i32), age.astype(i32),
           occup.astype(i32), zipc.astype(i32), item_id.astype(i32) - 1,
           year.astype(i32))
    cidx = jnp.concatenate([v + o for v, o in zip(ids, OFFS)], axis=0)

    w1p = jnp.pad(W1, ((0, 0), (0, H1P - 100)))
    b1p = jnp.pad(b1, (0, H1P - 100)).reshape(1, H1P)
    w2p = jnp.pad(W2, ((0, H1P - 100), (0, H2P - 50)))
    b2p = jnp.zeros((H2P,), f32).at[:50].set(b2).at[50].set(1.0)
    b2p = b2p.reshape(1, H2P)
    w3p = jnp.zeros((H2P,), f32).at[:50].set(W3[:, 0]).at[50].set(b3[0])
    w3p = w3p.reshape(1, H2P)

    proj = pl.pallas_call(
        _proj_body,
        out_shape=jax.ShapeDtypeStruct((RTOT, H1P), f32),
    )(stacked, w1p)

    hpre = _make_gather_sum()(proj, cidx)

    rb = 2048
    grid = B // rb
    out = pl.pallas_call(
        _tail_body,
        grid=(grid,),
        in_specs=[
            pl.BlockSpec((rb, H1P), lambda i: (i, 0)),
            pl.BlockSpec((1, H1P), lambda i: (0, 0)),
            pl.BlockSpec((H1P, H2P), lambda i: (0, 0)),
            pl.BlockSpec((1, H2P), lambda i: (0, 0)),
            pl.BlockSpec((1, H2P), lambda i: (0, 0)),
        ],
        out_specs=pl.BlockSpec((rb,), lambda i: (i,)),
        out_shape=jax.ShapeDtypeStruct((B,), f32),
    )(hpre, b1p, w2p, b2p, w3p)
    return out


# speedup vs baseline: 5.5849x; 5.4926x over previous
import functools

import jax
import jax.numpy as jnp
from jax import lax
from jax.experimental import pallas as pl
from jax.experimental.pallas import tpu as pltpu
from jax.experimental.pallas import tpu_sc as plsc

B = 16384
EMB = 128
H1P = 128
H2P = 128

BIG_SIZES = (6041, 3439, 3884)
BIG_PADS = (6048, 3440, 3888)
BIG_OFFS = (0, 6048, 9488)
BIG_W1 = (0, 4, 5)
RBIG = 13376

SMALL_SIZES = (2, 7, 21, 81)
SMALL_PADS = (8, 8, 24, 88)
SMALL_OFFS = (0, 8, 16, 40)
SMALL_W1 = (1, 2, 3, 6)
SBLK = 128

NC = 2
NS = 16
NW = NC * NS
BPW = B // NW
CH = 128
NCH = BPW // CH


def _proj_body(tb_ref, ts_ref, w_ref, pb_ref, ps_ref):
    for o, n, wi in zip(BIG_OFFS, BIG_PADS, BIG_W1):
        pb_ref[o:o + n, :] = jnp.dot(
            tb_ref[o:o + n, :], w_ref[wi * EMB:(wi + 1) * EMB, :],
            preferred_element_type=jnp.float32)
    for o, n, wi in zip(SMALL_OFFS, SMALL_PADS, SMALL_W1):
        ps_ref[o:o + n, :] = jnp.dot(
            ts_ref[o:o + n, :], w_ref[wi * EMB:(wi + 1) * EMB, :],
            preferred_element_type=jnp.float32)


def _tail_body(h_ref, si_ref, sb_ref, b1_ref, w2_ref, b2_ref, w3_ref, o_ref):
    rb = h_ref.shape[0]
    iota = lax.broadcasted_iota(jnp.int32, (rb, SBLK), 1)
    mh = jnp.zeros((rb, SBLK), jnp.float32)
    for f in range(4):
        mh = mh + (si_ref[f, :].reshape(rb, 1) == iota).astype(jnp.float32)
    x = (h_ref[...] + b1_ref[...]
         + jnp.dot(mh, sb_ref[...], preferred_element_type=jnp.float32))
    x = jnp.maximum(x, 0.0)
    h2 = jnp.maximum(
        jnp.dot(x, w2_ref[...], preferred_element_type=jnp.float32)
        + b2_ref[...], 0.0)
    o_ref[...] = jnp.sum(h2 * w3_ref[...], axis=1)


def _gather3_body(p_hbm, idx_hbm, out_hbm, idx_v,
                  b00, b01, b02, b10, b11, b12, sem0, sem1):
    wid = lax.axis_index("s") * NC + lax.axis_index("c")
    base = wid * BPW
    pltpu.sync_copy(idx_hbm.at[pl.ds(wid * 3 * BPW, 3 * BPW)], idx_v)
    sets = ((b00, b01, b02, sem0), (b10, b11, b12, sem1))

    def fire(ch, setn):
        bufs = sets[setn]
        sem = bufs[3]
        waits = []
        for f in range(3):
            isl = idx_v.at[pl.ds(f * BPW + ch * CH, CH)]
            waits.append(pltpu.async_copy(p_hbm.at[isl], bufs[f], sem))
        return waits

    pend = fire(0, 0)
    for ch in range(NCH):
        if ch + 1 < NCH:
            nxt = fire(ch + 1, (ch + 1) % 2)
        else:
            nxt = None
        for w in pend:
            w.wait()
        b0, b1_, b2_, _ = sets[ch % 2]

        def _sum_rows(r, _):
            for rr in range(2):
                row = r * 2 + rr
                for c in range(H1P // 16):
                    sl = pl.ds(c * 16, 16)
                    plsc.addupdate(b0.at[row, sl],
                                   b1_[row, sl] + b2_[row, sl])
            return 0

        lax.fori_loop(0, CH // 2, _sum_rows, 0)
        pltpu.sync_copy(b0, out_hbm.at[pl.ds(base + ch * CH, CH)])
        pend = nxt


@functools.cache
def _make_gather3():
    mesh = plsc.VectorSubcoreMesh(core_axis_name="c", subcore_axis_name="s",
                                  num_cores=NC, num_subcores=NS)
    buf = pltpu.VMEM((CH, H1P), jnp.float32)
    return functools.partial(
        pl.kernel,
        out_type=jax.ShapeDtypeStruct((B, H1P), jnp.float32),
        mesh=mesh,
        scratch_types=[
            pltpu.VMEM((3 * BPW,), jnp.int32),
            buf, buf, buf, buf, buf, buf,
            pltpu.SemaphoreType.DMA,
            pltpu.SemaphoreType.DMA,
        ],
    )(_gather3_body)


def kernel(user_id, gender, age, occup, zipc, item_id, year,
           user_table, gender_table, age_table, occup_table, zip_table,
           item_table, year_table, W1, b1, W2, b2, W3, b3):
    f32 = jnp.float32
    i32 = jnp.int32

    big_tabs = (user_table, zip_table, item_table)
    stacked_big = jnp.concatenate(
        [jnp.pad(t, ((0, p - s), (0, 0)))
         for t, s, p in zip(big_tabs, BIG_SIZES, BIG_PADS)], axis=0)
    small_tabs = (gender_table, age_table, occup_table, year_table)
    stacked_small = jnp.concatenate(
        [jnp.pad(t, ((0, p - s), (0, 0)))
         for t, s, p in zip(small_tabs, SMALL_SIZES, SMALL_PADS)], axis=0)

    big_ids = jnp.stack([
        user_id.astype(i32) - 1,
        zipc.astype(i32) + BIG_OFFS[1],
        item_id.astype(i32) - 1 + BIG_OFFS[2],
    ])
    bidx = big_ids.reshape(3, NW, BPW).transpose(1, 0, 2).reshape(-1)

    sidx = jnp.stack([
        gender.astype(i32) + SMALL_OFFS[0],
        age.astype(i32) + SMALL_OFFS[1],
        occup.astype(i32) + SMALL_OFFS[2],
        year.astype(i32) + SMALL_OFFS[3],
    ])

    w1p = jnp.pad(W1, ((0, 0), (0, H1P - 100)))
    b1p = jnp.pad(b1, (0, H1P - 100)).reshape(1, H1P)
    w2p = jnp.pad(W2, ((0, H1P - 100), (0, H2P - 50)))
    b2p = jnp.zeros((H2P,), f32).at[:50].set(b2).at[50].set(1.0)
    b2p = b2p.reshape(1, H2P)
    w3p = jnp.zeros((H2P,), f32).at[:50].set(W3[:, 0]).at[50].set(b3[0])
    w3p = w3p.reshape(1, H2P)

    p_big, p_small = pl.pallas_call(
        _proj_body,
        out_shape=(jax.ShapeDtypeStruct((RBIG, H1P), f32),
                   jax.ShapeDtypeStruct((SBLK, H1P), f32)),
    )(stacked_big, stacked_small, w1p)

    hpre = _make_gather3()(p_big, bidx)

    rb = 2048
    out = pl.pallas_call(
        _tail_body,
        grid=(B // rb,),
        in_specs=[
            pl.BlockSpec((rb, H1P), lambda i: (i, 0)),
            pl.BlockSpec((4, rb), lambda i: (0, i)),
            pl.BlockSpec((SBLK, H1P), lambda i: (0, 0)),
            pl.BlockSpec((1, H1P), lambda i: (0, 0)),
            pl.BlockSpec((H1P, H2P), lambda i: (0, 0)),
            pl.BlockSpec((1, H2P), lambda i: (0, 0)),
            pl.BlockSpec((1, H2P), lambda i: (0, 0)),
        ],
        out_specs=pl.BlockSpec((rb,), lambda i: (i,)),
        out_shape=jax.ShapeDtypeStruct((B,), f32),
    )(hpre, sidx, p_small, b1p, w2p, b2p, w3p)
    return out


# speedup vs baseline: 7.3370x; 1.3137x over previous
import functools

import jax
import jax.numpy as jnp
from jax import lax
from jax.experimental import pallas as pl
from jax.experimental.pallas import tpu as pltpu
from jax.experimental.pallas import tpu_sc as plsc

B = 16384
EMB = 128
H1P = 128
H2P = 128

BIG_SIZES = (6041, 3439, 3884)
BIG_OFFS = (0, 6048, 9488)
BIG_W1 = (0, 4, 5)
RBIG = 13376
BIG_ADJ = (-1, 6048, 9487)

SMALL_SIZES = (2, 7, 21, 81)
SMALL_OFFS = (0, 8, 16, 40)
SMALL_W1 = (1, 2, 3, 6)
SBLK = 128

NC = 2
NS = 16
NW = NC * NS
BPW = B // NW
CH = 128
NCH = BPW // CH


def _proj_body(ut, zt, it, gt, at_, ot, yt, w_ref, pb_ref, ps_ref):
    for ref, o, wi in zip((ut, zt, it), BIG_OFFS, BIG_W1):
        n = ref.shape[0]
        pb_ref[o:o + n, :] = jnp.dot(
            ref[...], w_ref[wi * EMB:(wi + 1) * EMB, :],
            preferred_element_type=jnp.float32)
    ps_ref[...] = jnp.zeros((SBLK, H1P), jnp.float32)
    for ref, o, wi in zip((gt, at_, ot, yt), SMALL_OFFS, SMALL_W1):
        n = ref.shape[0]
        ps_ref[o:o + n, :] = jnp.dot(
            ref[...], w_ref[wi * EMB:(wi + 1) * EMB, :],
            preferred_element_type=jnp.float32)


def _small_body(si_ref, sb_ref, o_ref):
    rb = o_ref.shape[0]
    iota = lax.broadcasted_iota(jnp.int32, (rb, SBLK), 1)
    mh = jnp.zeros((rb, SBLK), jnp.float32)
    for f in range(4):
        mh = mh + (si_ref[f, :].reshape(rb, 1) == iota).astype(jnp.float32)
    o_ref[...] = jnp.dot(mh, sb_ref[...], preferred_element_type=jnp.float32)


def _tail_body(h_ref, ss_ref, b1_ref, w2_ref, b2_ref, w3_ref, o_ref):
    x = jnp.maximum(h_ref[...] + ss_ref[...] + b1_ref[...], 0.0)
    h2 = jnp.maximum(
        jnp.dot(x, w2_ref[...], preferred_element_type=jnp.float32)
        + b2_ref[...], 0.0)
    o_ref[...] = jnp.sum(h2 * w3_ref[...], axis=1)


def _gather3_body(p_hbm, uid_hbm, zid_hbm, iid_hbm, out_hbm, idx_v,
                  b00, b01, b02, b10, b11, b12, sem0, sem1):
    wid = lax.axis_index("s") * NC + lax.axis_index("c")
    base = wid * BPW
    for f, ids in enumerate((uid_hbm, zid_hbm, iid_hbm)):
        pltpu.sync_copy(ids.at[pl.ds(base, BPW)],
                        idx_v.at[pl.ds(f * BPW, BPW)])

    def _adjust(i, _):
        for f in range(3):
            sl = pl.ds(f * BPW + i * 16, 16)
            idx_v[sl] = idx_v[sl] + BIG_ADJ[f]
        return 0

    lax.fori_loop(0, BPW // 16, _adjust, 0)

    sets = ((b00, b01, b02, sem0), (b10, b11, b12, sem1))

    def fire(ch, setn):
        bufs = sets[setn]
        sem = bufs[3]
        return [pltpu.async_copy(
                    p_hbm.at[idx_v.at[pl.ds(f * BPW + ch * CH, CH)]],
                    bufs[f], sem)
                for f in range(3)]

    pend = fire(0, 0)
    for ch in range(NCH):
        nxt = fire(ch + 1, (ch + 1) % 2) if ch + 1 < NCH else None
        for w in pend:
            w.wait()
        b0, b1_, b2_, _ = sets[ch % 2]

        def _sum_rows(r, _):
            for rr in range(2):
                row = r * 2 + rr
                for c in range(H1P // 16):
                    sl = pl.ds(c * 16, 16)
                    plsc.addupdate(b0.at[row, sl],
                                   b1_[row, sl] + b2_[row, sl])
            return 0

        lax.fori_loop(0, CH // 2, _sum_rows, 0)
        pltpu.sync_copy(b0, out_hbm.at[pl.ds(base + ch * CH, CH)])
        pend = nxt


@functools.cache
def _make_gather3():
    mesh = plsc.VectorSubcoreMesh(core_axis_name="c", subcore_axis_name="s",
                                  num_cores=NC, num_subcores=NS)
    buf = pltpu.VMEM((CH, H1P), jnp.float32)
    return functools.partial(
        pl.kernel,
        out_type=jax.ShapeDtypeStruct((B, H1P), jnp.float32),
        mesh=mesh,
        scratch_types=[
            pltpu.VMEM((3 * BPW,), jnp.int32),
            buf, buf, buf, buf, buf, buf,
            pltpu.SemaphoreType.DMA,
            pltpu.SemaphoreType.DMA,
        ],
    )(_gather3_body)


def kernel(user_id, gender, age, occup, zipc, item_id, year,
           user_table, gender_table, age_table, occup_table, zip_table,
           item_table, year_table, W1, b1, W2, b2, W3, b3):
    f32 = jnp.float32
    i32 = jnp.int32

    sidx = jnp.stack([
        gender.astype(i32) + SMALL_OFFS[0],
        age.astype(i32) + SMALL_OFFS[1],
        occup.astype(i32) + SMALL_OFFS[2],
        year.astype(i32) + SMALL_OFFS[3],
    ])

    w1p = jnp.pad(W1, ((0, 0), (0, H1P - 100)))
    b1p = jnp.pad(b1, (0, H1P - 100)).reshape(1, H1P)
    w2p = jnp.pad(W2, ((0, H1P - 100), (0, H2P - 50)))
    b2p = jnp.zeros((H2P,), f32).at[:50].set(b2).at[50].set(1.0)
    b2p = b2p.reshape(1, H2P)
    w3p = jnp.zeros((H2P,), f32).at[:50].set(W3[:, 0]).at[50].set(b3[0])
    w3p = w3p.reshape(1, H2P)

    p_big, p_small = pl.pallas_call(
        _proj_body,
        out_shape=(jax.ShapeDtypeStruct((RBIG, H1P), f32),
                   jax.ShapeDtypeStruct((SBLK, H1P), f32)),
    )(user_table, zip_table, item_table, gender_table, age_table,
      occup_table, year_table, w1p)

    hpre = _make_gather3()(p_big, user_id.astype(i32), zipc.astype(i32),
                           item_id.astype(i32))

    rb = 2048
    ssum = pl.pallas_call(
        _small_body,
        grid=(B // rb,),
        in_specs=[
            pl.BlockSpec((4, rb), lambda i: (0, i)),
            pl.BlockSpec((SBLK, H1P), lambda i: (0, 0)),
        ],
        out_specs=pl.BlockSpec((rb, H1P), lambda i: (i, 0)),
        out_shape=jax.ShapeDtypeStruct((B, H1P), f32),
    )(sidx, p_small)

    out = pl.pallas_call(
        _tail_body,
        grid=(B // rb,),
        in_specs=[
            pl.BlockSpec((rb, H1P), lambda i: (i, 0)),
            pl.BlockSpec((rb, H1P), lambda i: (i, 0)),
            pl.BlockSpec((1, H1P), lambda i: (0, 0)),
            pl.BlockSpec((H1P, H2P), lambda i: (0, 0)),
            pl.BlockSpec((1, H2P), lambda i: (0, 0)),
            pl.BlockSpec((1, H2P), lambda i: (0, 0)),
        ],
        out_specs=pl.BlockSpec((rb,), lambda i: (i,)),
        out_shape=jax.ShapeDtypeStruct((B,), f32),
    )(hpre, ssum, b1p, w2p, b2p, w3p)
    return out


# speedup vs baseline: 7.6934x; 1.0486x over previous
import functools

import jax
import jax.numpy as jnp
from jax import lax
from jax.experimental import pallas as pl
from jax.experimental.pallas import tpu as pltpu
from jax.experimental.pallas import tpu_sc as plsc

B = 16384
EMB = 128
H1P = 128
H2P = 128

BIG_SIZES = (6041, 3439, 3884)
BIG_PADS = (6048, 3440, 3888)
BIG_W1 = (0, 4, 5)
BIG_ADJ = (-1, 0, -1)

SMALL_SIZES = (2, 7, 21, 81)
SMALL_OFFS = (0, 8, 16, 40)
SMALL_W1 = (1, 2, 3, 6)
SBLK = 128

NC = 2
NS = 16
NW = NC * NS
BPW = B // NW
CH = 128
NCH = BPW // CH


def _proj_body(ut, zt, it, gt, at_, ot, yt, w_ref, pu, pz, pi, ps_ref):
    for ref, out, wi in zip((ut, zt, it), (pu, pz, pi), BIG_W1):
        out[...] = jnp.dot(ref[...], w_ref[wi * EMB:(wi + 1) * EMB, :],
                           preferred_element_type=jnp.float32)
    @pl.when(pl.program_id(0) == 0)
    def _():
        ps_ref[...] = jnp.zeros((SBLK, H1P), jnp.float32)
        for ref, o, wi in zip((gt, at_, ot, yt), SMALL_OFFS, SMALL_W1):
            n = ref.shape[0]
            ps_ref[o:o + n, :] = jnp.dot(
                ref[...], w_ref[wi * EMB:(wi + 1) * EMB, :],
                preferred_element_type=jnp.float32)


def _small_body(si_ref, sb_ref, o_ref):
    rb = o_ref.shape[0]
    iota = lax.broadcasted_iota(jnp.int32, (rb, SBLK), 1)
    mh = jnp.zeros((rb, SBLK), jnp.float32)
    for f in range(4):
        mh = mh + (si_ref[f, :].reshape(rb, 1) == iota).astype(jnp.float32)
    o_ref[...] = jnp.dot(mh, sb_ref[...], preferred_element_type=jnp.float32)


def _tail_body(h_ref, ss_ref, b1_ref, w2_ref, b2_ref, w3_ref, o_ref):
    x = jnp.maximum(h_ref[...] + ss_ref[...] + b1_ref[...], 0.0)
    h2 = jnp.maximum(
        jnp.dot(x, w2_ref[...], preferred_element_type=jnp.float32)
        + b2_ref[...], 0.0)
    o_ref[...] = jnp.sum(h2 * w3_ref[...], axis=1)


def _gather3_body(pu_hbm, pz_hbm, pi_hbm, uid_hbm, zid_hbm, iid_hbm,
                  out_hbm, idx_v, b00, b01, b02, b10, b11, b12,
                  sem0, sem1, isem):
    wid = lax.axis_index("s") * NC + lax.axis_index("c")
    base = wid * BPW
    tabs = (pu_hbm, pz_hbm, pi_hbm)
    iw = []
    for f, ids in enumerate((uid_hbm, zid_hbm, iid_hbm)):
        iw.append(pltpu.async_copy(ids.at[pl.ds(base, BPW)],
                                   idx_v.at[pl.ds(f * BPW, BPW)], isem))
    for w in iw:
        w.wait()

    def _adjust(i, _):
        for f in (0, 2):
            sl = pl.ds(f * BPW + i * 16, 16)
            idx_v[sl] = idx_v[sl] + BIG_ADJ[f]
        return 0

    lax.fori_loop(0, BPW // 16, _adjust, 0)

    sets = ((b00, b01, b02, sem0), (b10, b11, b12, sem1))

    def fire(ch, setn):
        bufs = sets[setn]
        sem = bufs[3]
        return [pltpu.async_copy(
                    tabs[f].at[idx_v.at[pl.ds(f * BPW + ch * CH, CH)]],
                    bufs[f], sem)
                for f in range(3)]

    pend = fire(0, 0)
    for ch in range(NCH):
        nxt = fire(ch + 1, (ch + 1) % 2) if ch + 1 < NCH else None
        for w in pend:
            w.wait()
        b0, b1_, b2_, _ = sets[ch % 2]

        def _sum_rows(r, _):
            for rr in range(2):
                row = r * 2 + rr
                for c in range(H1P // 16):
                    sl = pl.ds(c * 16, 16)
                    plsc.addupdate(b0.at[row, sl],
                                   b1_[row, sl] + b2_[row, sl])
            return 0

        lax.fori_loop(0, CH // 2, _sum_rows, 0)
        pltpu.sync_copy(b0, out_hbm.at[pl.ds(base + ch * CH, CH)])
        pend = nxt


@functools.cache
def _make_gather3():
    mesh = plsc.VectorSubcoreMesh(core_axis_name="c", subcore_axis_name="s",
                                  num_cores=NC, num_subcores=NS)
    buf = pltpu.VMEM((CH, H1P), jnp.float32)
    return functools.partial(
        pl.kernel,
        out_type=jax.ShapeDtypeStruct((B, H1P), jnp.float32),
        mesh=mesh,
        scratch_types=[
            pltpu.VMEM((3 * BPW,), jnp.int32),
            buf, buf, buf, buf, buf, buf,
            pltpu.SemaphoreType.DMA,
            pltpu.SemaphoreType.DMA,
            pltpu.SemaphoreType.DMA,
        ],
    )(_gather3_body)


def kernel(user_id, gender, age, occup, zipc, item_id, year,
           user_table, gender_table, age_table, occup_table, zip_table,
           item_table, year_table, W1, b1, W2, b2, W3, b3):
    f32 = jnp.float32
    i32 = jnp.int32

    sidx = jnp.stack([
        gender.astype(i32) + SMALL_OFFS[0],
        age.astype(i32) + SMALL_OFFS[1],
        occup.astype(i32) + SMALL_OFFS[2],
        year.astype(i32) + SMALL_OFFS[3],
    ])

    w1p = jnp.pad(W1, ((0, 0), (0, H1P - 100)))
    b1p = jnp.pad(b1, (0, H1P - 100)).reshape(1, H1P)
    w2p = jnp.pad(W2, ((0, H1P - 100), (0, H2P - 50)))
    b2p = jnp.zeros((H2P,), f32).at[:50].set(b2).at[50].set(1.0)
    b2p = b2p.reshape(1, H2P)
    w3p = jnp.zeros((H2P,), f32).at[:50].set(W3[:, 0]).at[50].set(b3[0])
    w3p = w3p.reshape(1, H2P)

    halves = tuple(p // 2 for p in BIG_PADS)

    p_user, p_zip, p_item, p_small = pl.pallas_call(
        _proj_body,
        grid=(2,),
        in_specs=[
            pl.BlockSpec((halves[0], EMB), lambda i: (i, 0)),
            pl.BlockSpec((halves[1], EMB), lambda i: (i, 0)),
            pl.BlockSpec((halves[2], EMB), lambda i: (i, 0)),
            pl.BlockSpec((SMALL_SIZES[0], EMB), lambda i: (0, 0)),
            pl.BlockSpec((SMALL_SIZES[1], EMB), lambda i: (0, 0)),
            pl.BlockSpec((SMALL_SIZES[2], EMB), lambda i: (0, 0)),
            pl.BlockSpec((SMALL_SIZES[3], EMB), lambda i: (0, 0)),
            pl.BlockSpec((7 * EMB, H1P), lambda i: (0, 0)),
        ],
        out_specs=(
            pl.BlockSpec((halves[0], H1P), lambda i: (i, 0)),
            pl.BlockSpec((halves[1], H1P), lambda i: (i, 0)),
            pl.BlockSpec((halves[2], H1P), lambda i: (i, 0)),
            pl.BlockSpec((SBLK, H1P), lambda i: (0, 0)),
        ),
        out_shape=(jax.ShapeDtypeStruct((BIG_PADS[0], H1P), f32),
                   jax.ShapeDtypeStruct((BIG_PADS[1], H1P), f32),
                   jax.ShapeDtypeStruct((BIG_PADS[2], H1P), f32),
                   jax.ShapeDtypeStruct((SBLK, H1P), f32)),
    )(user_table, zip_table, item_table, gender_table, age_table,
      occup_table, year_table, w1p)

    hpre = _make_gather3()(p_user, p_zip, p_item, user_id.astype(i32),
                           zipc.astype(i32), item_id.astype(i32))

    rb = 2048
    ssum = pl.pallas_call(
        _small_body,
        grid=(B // rb,),
        in_specs=[
            pl.BlockSpec((4, rb), lambda i: (0, i)),
            pl.BlockSpec((SBLK, H1P), lambda i: (0, 0)),
        ],
        out_specs=pl.BlockSpec((rb, H1P), lambda i: (i, 0)),
        out_shape=jax.ShapeDtypeStruct((B, H1P), f32),
    )(sidx, p_small)

    rbt = 4096
    out = pl.pallas_call(
        _tail_body,
        grid=(B // rbt,),
        in_specs=[
            pl.BlockSpec((rbt, H1P), lambda i: (i, 0)),
            pl.BlockSpec((rbt, H1P), lambda i: (i, 0)),
            pl.BlockSpec((1, H1P), lambda i: (0, 0)),
            pl.BlockSpec((H1P, H2P), lambda i: (0, 0)),
            pl.BlockSpec((1, H2P), lambda i: (0, 0)),
            pl.BlockSpec((1, H2P), lambda i: (0, 0)),
        ],
        out_specs=pl.BlockSpec((rbt,), lambda i: (i,)),
        out_shape=jax.ShapeDtypeStruct((B,), f32),
    )(hpre, ssum, b1p, w2p, b2p, w3p)
    return out


# speedup vs baseline: 7.7577x; 1.0084x over previous
import functools

import jax
import jax.numpy as jnp
from jax import lax
from jax.experimental import pallas as pl
from jax.experimental.pallas import tpu as pltpu
from jax.experimental.pallas import tpu_sc as plsc

B = 16384
EMB = 128
H1P = 128
H2P = 128

BIG_SIZES = (6041, 3439, 3884)
BIG_PADS = (6048, 3440, 3888)
BIG_W1 = (0, 4, 5)
BIG_ADJ = (-1, 0, -1)

SMALL_SIZES = (2, 7, 21, 81)
SMALL_OFFS = (0, 8, 16, 40)
SMALL_W1 = (1, 2, 3, 6)
SBLK = 128

NC = 2
NS = 16
NW = NC * NS
BPW = B // NW
CH = 128
NCH = BPW // CH


def _proj_body(ut, zt, it, gt, at_, ot, yt, w_ref, pu, pz, pi, ps_ref):
    for ref, out, wi in zip((ut, zt, it), (pu, pz, pi), BIG_W1):
        out[...] = jnp.dot(ref[...], w_ref[wi * EMB:(wi + 1) * EMB, :],
                           preferred_element_type=jnp.float32)
    @pl.when(pl.program_id(0) == 0)
    def _():
        ps_ref[...] = jnp.zeros((SBLK, H1P), jnp.float32)
        for ref, o, wi in zip((gt, at_, ot, yt), SMALL_OFFS, SMALL_W1):
            n = ref.shape[0]
            ps_ref[o:o + n, :] = jnp.dot(
                ref[...], w_ref[wi * EMB:(wi + 1) * EMB, :],
                preferred_element_type=jnp.float32)


def _small_body(g_ref, a_ref, o_ref_, y_ref, sb_ref, o_ref):
    rb = o_ref.shape[0]
    iota = lax.broadcasted_iota(jnp.int32, (rb, SBLK), 1)
    mh = jnp.zeros((rb, SBLK), jnp.float32)
    for ref, off in zip((g_ref, a_ref, o_ref_, y_ref), SMALL_OFFS):
        mh = mh + (ref[0, :].reshape(rb, 1) == iota - off).astype(jnp.float32)
    o_ref[...] = jnp.dot(mh, sb_ref[...], preferred_element_type=jnp.float32)


def _tail_body(h_ref, ss_ref, b1_ref, w2_ref, b2_ref, w3_ref, o_ref):
    x = jnp.maximum(h_ref[...] + ss_ref[...] + b1_ref[...], 0.0)
    h2 = jnp.maximum(
        jnp.dot(x, w2_ref[...], preferred_element_type=jnp.float32)
        + b2_ref[...], 0.0)
    o_ref[...] = jnp.sum(h2 * w3_ref[...], axis=1)


def _gather3_body(pu_hbm, pz_hbm, pi_hbm, uid_hbm, zid_hbm, iid_hbm,
                  out_hbm, idx_v, b00, b01, b02, b10, b11, b12,
                  sem0, sem1, osem0, osem1, isem):
    wid = lax.axis_index("s") * NC + lax.axis_index("c")
    base = wid * BPW
    tabs = (pu_hbm, pz_hbm, pi_hbm)
    iw = []
    for f, ids in enumerate((uid_hbm, zid_hbm, iid_hbm)):
        iw.append(pltpu.async_copy(ids.at[pl.ds(base, BPW)],
                                   idx_v.at[pl.ds(f * BPW, BPW)], isem))
    for w in iw:
        w.wait()

    def _make_adjust(lo):
        def _adjust(i, _):
            for f in (0, 2):
                sl = pl.ds(f * BPW + lo + i * 16, 16)
                idx_v[sl] = idx_v[sl] + BIG_ADJ[f]
            return 0
        return _adjust

    lax.fori_loop(0, CH // 16, _make_adjust(0), 0)

    sets = ((b00, b01, b02, sem0), (b10, b11, b12, sem1))
    osems = (osem0, osem1)

    def fire(ch, setn):
        bufs = sets[setn]
        sem = bufs[3]
        return [pltpu.async_copy(
                    tabs[f].at[idx_v.at[pl.ds(f * BPW + ch * CH, CH)]],
                    bufs[f], sem)
                for f in range(3)]

    pend = fire(0, 0)
    lax.fori_loop(0, (BPW - CH) // 16, _make_adjust(CH), 0)
    owait = [None, None]
    for ch in range(NCH):
        if ch + 1 < NCH:
            s1 = (ch + 1) % 2
            if owait[s1] is not None:
                owait[s1].wait()
                owait[s1] = None
            nxt = fire(ch + 1, s1)
        else:
            nxt = None
        for w in pend:
            w.wait()
        b0, b1_, b2_, _ = sets[ch % 2]

        def _sum_rows(r, _):
            for rr in range(2):
                row = r * 2 + rr
                for c in range(H1P // 16):
                    sl = pl.ds(c * 16, 16)
                    plsc.addupdate(b0.at[row, sl],
                                   b1_[row, sl] + b2_[row, sl])
            return 0

        lax.fori_loop(0, CH // 2, _sum_rows, 0)
        owait[ch % 2] = pltpu.async_copy(
            b0, out_hbm.at[pl.ds(base + ch * CH, CH)], osems[ch % 2])
        pend = nxt
    for s in (0, 1):
        if owait[s] is not None:
            owait[s].wait()


@functools.cache
def _make_gather3():
    mesh = plsc.VectorSubcoreMesh(core_axis_name="c", subcore_axis_name="s",
                                  num_cores=NC, num_subcores=NS)
    buf = pltpu.VMEM((CH, H1P), jnp.float32)
    return functools.partial(
        pl.kernel,
        out_type=jax.ShapeDtypeStruct((B, H1P), jnp.float32),
        mesh=mesh,
        scratch_types=[
            pltpu.VMEM((3 * BPW,), jnp.int32),
            buf, buf, buf, buf, buf, buf,
            pltpu.SemaphoreType.DMA,
            pltpu.SemaphoreType.DMA,
            pltpu.SemaphoreType.DMA,
            pltpu.SemaphoreType.DMA,
            pltpu.SemaphoreType.DMA,
        ],
    )(_gather3_body)


def kernel(user_id, gender, age, occup, zipc, item_id, year,
           user_table, gender_table, age_table, occup_table, zip_table,
           item_table, year_table, W1, b1, W2, b2, W3, b3):
    f32 = jnp.float32
    i32 = jnp.int32

    w1p = jnp.pad(W1, ((0, 0), (0, H1P - 100)))
    b1p = jnp.pad(b1, (0, H1P - 100)).reshape(1, H1P)
    w2p = jnp.pad(W2, ((0, H1P - 100), (0, H2P - 50)))
    b2p = jnp.zeros((H2P,), f32).at[:50].set(b2).at[50].set(1.0)
    b2p = b2p.reshape(1, H2P)
    w3p = jnp.zeros((H2P,), f32).at[:50].set(W3[:, 0]).at[50].set(b3[0])
    w3p = w3p.reshape(1, H2P)

    halves = tuple(p // 2 for p in BIG_PADS)

    p_user, p_zip, p_item, p_small = pl.pallas_call(
        _proj_body,
        grid=(2,),
        in_specs=[
            pl.BlockSpec((halves[0], EMB), lambda i: (i, 0)),
            pl.BlockSpec((halves[1], EMB), lambda i: (i, 0)),
            pl.BlockSpec((halves[2], EMB), lambda i: (i, 0)),
            pl.BlockSpec((SMALL_SIZES[0], EMB), lambda i: (0, 0)),
            pl.BlockSpec((SMALL_SIZES[1], EMB), lambda i: (0, 0)),
            pl.BlockSpec((SMALL_SIZES[2], EMB), lambda i: (0, 0)),
            pl.BlockSpec((SMALL_SIZES[3], EMB), lambda i: (0, 0)),
            pl.BlockSpec((7 * EMB, H1P), lambda i: (0, 0)),
        ],
        out_specs=(
            pl.BlockSpec((halves[0], H1P), lambda i: (i, 0)),
            pl.BlockSpec((halves[1], H1P), lambda i: (i, 0)),
            pl.BlockSpec((halves[2], H1P), lambda i: (i, 0)),
            pl.BlockSpec((SBLK, H1P), lambda i: (0, 0)),
        ),
        out_shape=(jax.ShapeDtypeStruct((BIG_PADS[0], H1P), f32),
                   jax.ShapeDtypeStruct((BIG_PADS[1], H1P), f32),
                   jax.ShapeDtypeStruct((BIG_PADS[2], H1P), f32),
                   jax.ShapeDtypeStruct((SBLK, H1P), f32)),
    )(user_table, zip_table, item_table, gender_table, age_table,
      occup_table, year_table, w1p)

    hpre = _make_gather3()(p_user, p_zip, p_item, user_id.astype(i32),
                           zipc.astype(i32), item_id.astype(i32))

    rb = 2048
    idspec = pl.BlockSpec((1, rb), lambda i: (0, i))
    ssum = pl.pallas_call(
        _small_body,
        grid=(B // rb,),
        in_specs=[
            idspec, idspec, idspec, idspec,
            pl.BlockSpec((SBLK, H1P), lambda i: (0, 0)),
        ],
        out_specs=pl.BlockSpec((rb, H1P), lambda i: (i, 0)),
        out_shape=jax.ShapeDtypeStruct((B, H1P), f32),
    )(gender.astype(i32).reshape(1, B), age.astype(i32).reshape(1, B),
      occup.astype(i32).reshape(1, B), year.astype(i32).reshape(1, B),
      p_small)

    rbt = 4096
    out = pl.pallas_call(
        _tail_body,
        grid=(B // rbt,),
        in_specs=[
            pl.BlockSpec((rbt, H1P), lambda i: (i, 0)),
            pl.BlockSpec((rbt, H1P), lambda i: (i, 0)),
            pl.BlockSpec((1, H1P), lambda i: (0, 0)),
            pl.BlockSpec((H1P, H2P), lambda i: (0, 0)),
            pl.BlockSpec((1, H2P), lambda i: (0, 0)),
            pl.BlockSpec((1, H2P), lambda i: (0, 0)),
        ],
        out_specs=pl.BlockSpec((rbt,), lambda i: (i,)),
        out_shape=jax.ShapeDtypeStruct((B,), f32),
    )(hpre, ssum, b1p, w2p, b2p, w3p)
    return out
